# Initial kernel scaffold; baseline (speedup 1.0000x reference)
import jax
import jax.numpy as jnp
from jax.experimental import pallas as pl


def kernel(x, tables):
    raise NotImplementedError("write your pallas kernel here")


# speedup vs baseline: 1.5191x; 1.5191x over previous
import functools

import jax
import jax.numpy as jnp
from jax import lax
from jax.experimental import pallas as pl
from jax.experimental.pallas import tpu as pltpu
from jax.experimental.pallas import tpu_sc as plsc

BATCH = 16384
N_CONT = 13
N_CAT = 26
VOCAB = 100000
EMB = 64

NUM_CORES = 2
NUM_SUBCORES = 16
NW = NUM_CORES * NUM_SUBCORES
ROWS_PER_W = BATCH // NW
BLOCK = 64
N_BLOCKS = ROWS_PER_W // BLOCK


def _make_kernel():
    mesh = plsc.VectorSubcoreMesh(core_axis_name="c", subcore_axis_name="s")

    @functools.partial(
        pl.kernel,
        mesh=mesh,
        out_type=jax.ShapeDtypeStruct((N_CAT, BATCH, EMB), jnp.float32),
        scratch_types=[
            pltpu.VMEM((ROWS_PER_W,), jnp.int32),
            pltpu.VMEM((ROWS_PER_W, EMB), jnp.float32),
            pltpu.SemaphoreType.DMA,
        ],
    )
    def emb_kernel(xcat_hbm, table_hbm, out_hbm, idx_v, rows_v, sem):
        wid = lax.axis_index("s") * NUM_CORES + lax.axis_index("c")
        base = wid * ROWS_PER_W

        def per_feature(c, carry):
            pltpu.sync_copy(xcat_hbm.at[wid, c], idx_v)

            def per_block(b, carry2):
                for v in range(BLOCK // 16):
                    off = b * BLOCK + v * 16
                    vec = idx_v[pl.ds(off, 16)]
                    for k in range(16):
                        pltpu.async_copy(
                            table_hbm.at[c, vec[k]], rows_v.at[off + k], sem)

                def drain(k, carry3):
                    m = b * BLOCK + k
                    pltpu.make_async_copy(
                        table_hbm.at[c, 0], rows_v.at[m], sem).wait()
                    return carry3

                lax.fori_loop(0, BLOCK, drain, 0)
                return carry2

            lax.fori_loop(0, N_BLOCKS, per_block, 0)
            pltpu.sync_copy(rows_v, out_hbm.at[c, pl.ds(base, ROWS_PER_W), :])
            return carry

        lax.fori_loop(0, N_CAT, per_feature, 0)

    return emb_kernel


_emb_kernel = _make_kernel()


def kernel(x, tables):
    xcat = x[:, N_CONT:].reshape(NW, ROWS_PER_W, N_CAT).transpose(0, 2, 1)
    emb = _emb_kernel(xcat, tables)
    xcont = x[:, :N_CONT].astype(jnp.float32)
    return jnp.concatenate(
        [xcont, emb.transpose(1, 0, 2).reshape(BATCH, N_CAT * EMB)], axis=1)
